# Initial kernel scaffold; baseline (speedup 1.0000x reference)
import jax
import jax.numpy as jnp
from jax.experimental import pallas as pl


def kernel(h, edge_index, coord, msg_W0, msg_b0, msg_W1, msg_b1, node_W0, node_b0, node_W1, node_b1, node_W2, node_b2, coord_W0, coord_b0, coord_W1, coord_b1, coord_W2):
    raise NotImplementedError("write your pallas kernel here")


# speedup vs baseline: 1.3973x; 1.3973x over previous
import functools

import jax
import jax.numpy as jnp
from jax.experimental import pallas as pl
from jax.experimental.pallas import tpu as pltpu


def _silu(x):
    return x * jax.nn.sigmoid(x)


def _proj_body(h_ref, w0at_ref, w0bt_ref, b0_ref, ha_ref, hb_ref):
    h = h_ref[...]
    ha_ref[...] = jnp.dot(h, w0at_ref[...], preferred_element_type=jnp.float32) + b0_ref[...]
    hb_ref[...] = jnp.dot(h, w0bt_ref[...], preferred_element_type=jnp.float32)


def _node_proj(h, w0at, w0bt, b0row, block=2000):
    n, d = h.shape
    grid = n // block
    return pl.pallas_call(
        _proj_body,
        grid=(grid,),
        in_specs=[
            pl.BlockSpec((block, d), lambda i: (i, 0)),
            pl.BlockSpec((d, d), lambda i: (0, 0)),
            pl.BlockSpec((d, d), lambda i: (0, 0)),
            pl.BlockSpec((1, d), lambda i: (0, 0)),
        ],
        out_specs=[
            pl.BlockSpec((block, d), lambda i: (i, 0)),
            pl.BlockSpec((block, d), lambda i: (i, 0)),
        ],
        out_shape=[
            jax.ShapeDtypeStruct((n, d), jnp.float32),
            jax.ShapeDtypeStruct((n, d), jnp.float32),
        ],
    )(h, w0at, w0bt, b0row)


def _edge_body(g1_ref, g2_ref, cr_ref, cc_ref, w1t_ref, cw0t_ref, cw1t_ref,
               consts_ref, m_out, t_out):
    cd = cr_ref[...] - cc_ref[...]
    radial = jnp.sum(cd * cd, axis=1, keepdims=True)
    w0c = consts_ref[0:1, :]
    b1 = consts_ref[1:2, :]
    cb0 = consts_ref[2:3, :]
    cb1 = consts_ref[3:4, :]
    cw2 = consts_ref[4:5, :]
    pre = g1_ref[...] + g2_ref[...] + radial * w0c
    m0 = _silu(pre)
    m = _silu(jnp.dot(m0, w1t_ref[...], preferred_element_type=jnp.float32) + b1)
    t0 = _silu(jnp.dot(m, cw0t_ref[...], preferred_element_type=jnp.float32) + cb0)
    t1 = _silu(jnp.dot(t0, cw1t_ref[...], preferred_element_type=jnp.float32) + cb1)
    tt = jnp.sum(t1 * cw2, axis=1, keepdims=True)
    m_out[...] = m
    lane = jax.lax.broadcasted_iota(jnp.int32, cd.shape, 1)
    t_out[...] = cd * tt + (lane == 3).astype(jnp.float32)


def _edge_mlp(g1, g2, cr, cc, w1t, cw0t, cw1t, consts, block=2000):
    e, d = g1.shape
    grid = e // block
    return pl.pallas_call(
        _edge_body,
        grid=(grid,),
        in_specs=[
            pl.BlockSpec((block, d), lambda i: (i, 0)),
            pl.BlockSpec((block, d), lambda i: (i, 0)),
            pl.BlockSpec((block, 16), lambda i: (i, 0)),
            pl.BlockSpec((block, 16), lambda i: (i, 0)),
            pl.BlockSpec((d, d), lambda i: (0, 0)),
            pl.BlockSpec((d, d), lambda i: (0, 0)),
            pl.BlockSpec((d, d), lambda i: (0, 0)),
            pl.BlockSpec((8, d), lambda i: (0, 0)),
        ],
        out_specs=[
            pl.BlockSpec((block, d), lambda i: (i, 0)),
            pl.BlockSpec((block, 16), lambda i: (i, 0)),
        ],
        out_shape=[
            jax.ShapeDtypeStruct((e, d), jnp.float32),
            jax.ShapeDtypeStruct((e, 16), jnp.float32),
        ],
    )(g1, g2, cr, cc, w1t, cw0t, cw1t, consts)


def _node_body(h_ref, sm_ref, st_ref, c16_ref, w0at_ref, w0bt_ref, w1t_ref,
               w2t_ref, consts_ref, hout_ref, cout_ref):
    nb0 = consts_ref[0:1, :]
    nb1 = consts_ref[1:2, :]
    nb2 = consts_ref[2:3, :]
    st = st_ref[0] + st_ref[1]
    cnt = st[:, 3:4]
    inv = 1.0 / jnp.maximum(cnt, 1.0)
    agg = (sm_ref[0] + sm_ref[1]) * inv
    h = h_ref[...]
    a = _silu(jnp.dot(h, w0at_ref[...], preferred_element_type=jnp.float32)
              + jnp.dot(agg, w0bt_ref[...], preferred_element_type=jnp.float32)
              + nb0)
    a2 = _silu(jnp.dot(a, w1t_ref[...], preferred_element_type=jnp.float32) + nb1)
    outp = jnp.dot(a2, w2t_ref[...], preferred_element_type=jnp.float32) + nb2
    hout_ref[...] = h + outp
    cout_ref[...] = c16_ref[...] + st * inv


def _node_mlp(h, sm, st, c16, w0at, w0bt, w1t, w2t, consts, block=2000):
    n, d = h.shape
    grid = n // block
    return pl.pallas_call(
        _node_body,
        grid=(grid,),
        in_specs=[
            pl.BlockSpec((block, d), lambda i: (i, 0)),
            pl.BlockSpec((2, block, d), lambda i: (0, i, 0)),
            pl.BlockSpec((2, block, 16), lambda i: (0, i, 0)),
            pl.BlockSpec((block, 16), lambda i: (i, 0)),
            pl.BlockSpec((d, d), lambda i: (0, 0)),
            pl.BlockSpec((d, d), lambda i: (0, 0)),
            pl.BlockSpec((d, d), lambda i: (0, 0)),
            pl.BlockSpec((d, d), lambda i: (0, 0)),
            pl.BlockSpec((8, d), lambda i: (0, 0)),
        ],
        out_specs=[
            pl.BlockSpec((block, d), lambda i: (i, 0)),
            pl.BlockSpec((block, 16), lambda i: (i, 0)),
        ],
        out_shape=[
            jax.ShapeDtypeStruct((n, d), jnp.float32),
            jax.ShapeDtypeStruct((n, 16), jnp.float32),
        ],
    )(h, sm, st, c16, w0at, w0bt, w1t, w2t, consts)


def kernel(h, edge_index, coord, msg_W0, msg_b0, msg_W1, msg_b1,
           node_W0, node_b0, node_W1, node_b1, node_W2, node_b2,
           coord_W0, coord_b0, coord_W1, coord_b1, coord_W2):
    n, d = h.shape
    e = edge_index.shape[0]
    row = edge_index[:, 0]
    col = edge_index[:, 1]

    w0at = msg_W0[:, :d].T
    w0bt = msg_W0[:, d:2 * d].T
    w0c = msg_W0[:, 2 * d]
    zeros_row = jnp.zeros((1, d), jnp.float32)
    edge_consts = jnp.concatenate([
        w0c[None, :], msg_b1[None, :], coord_b0[None, :], coord_b1[None, :],
        coord_W2, zeros_row, zeros_row, zeros_row], axis=0)
    node_consts = jnp.concatenate([
        node_b0[None, :], node_b1[None, :], node_b2[None, :],
        zeros_row, zeros_row, zeros_row, zeros_row, zeros_row], axis=0)

    coord16 = jnp.pad(coord, ((0, 0), (0, 16 - coord.shape[1])))

    ha, hb = _node_proj(h, w0at, w0bt, msg_b0[None, :])

    g1 = ha[row]
    g2 = hb[col]
    cr = coord16[row]
    cc = coord16[col]

    m, t16 = _edge_mlp(g1, g2, cr, cc, msg_W1.T, coord_W0.T, coord_W1.T,
                       edge_consts)

    sm = jax.ops.segment_sum(m, row, num_segments=n)
    st = jax.ops.segment_sum(t16, row, num_segments=n)
    sm2 = jnp.stack([sm, jnp.zeros_like(sm)])
    st2 = jnp.stack([st, jnp.zeros_like(st)])

    h_out, c16_out = _node_mlp(h, sm2, st2, coord16, node_W0[:, :d].T,
                               node_W0[:, d:].T, node_W1.T, node_W2.T,
                               node_consts)
    return (h_out, c16_out[:, :3])


# speedup vs baseline: 2.7518x; 1.9694x over previous
import functools

import jax
import jax.numpy as jnp
from jax import lax
from jax.experimental import pallas as pl
from jax.experimental.pallas import tpu as pltpu
from jax.experimental.pallas import tpu_sc as plsc

_NC = 2
_NS = 16
_NW = _NC * _NS
_EB = 80


def _silu(x):
    return x * jax.nn.sigmoid(x)


def _proj_body(h_ref, w0at_ref, w0bt_ref, b0_ref, ha_ref, hb_ref):
    h = h_ref[...]
    ha_ref[...] = jnp.dot(h, w0at_ref[...], preferred_element_type=jnp.float32) + b0_ref[...]
    hb_ref[...] = jnp.dot(h, w0bt_ref[...], preferred_element_type=jnp.float32)


def _node_proj(h, w0at, w0bt, b0row, block=2000):
    n, d = h.shape
    grid = n // block
    return pl.pallas_call(
        _proj_body,
        grid=(grid,),
        in_specs=[
            pl.BlockSpec((block, d), lambda i: (i, 0)),
            pl.BlockSpec((d, d), lambda i: (0, 0)),
            pl.BlockSpec((d, d), lambda i: (0, 0)),
            pl.BlockSpec((1, d), lambda i: (0, 0)),
        ],
        out_specs=[
            pl.BlockSpec((block, d), lambda i: (i, 0)),
            pl.BlockSpec((block, d), lambda i: (i, 0)),
        ],
        out_shape=[
            jax.ShapeDtypeStruct((n, d), jnp.float32),
            jax.ShapeDtypeStruct((n, d), jnp.float32),
        ],
    )(h, w0at, w0bt, b0row)


def _sc_gather(ha, hb, cw, row, col):
    e = row.shape[0]
    d = ha.shape[1]
    per_w = e // _NW
    iters = per_w // _EB
    mesh = plsc.VectorSubcoreMesh(core_axis_name="c", subcore_axis_name="s")

    @functools.partial(
        pl.kernel,
        out_type=[
            jax.ShapeDtypeStruct((e, d), jnp.float32),
            jax.ShapeDtypeStruct((e, d), jnp.float32),
            jax.ShapeDtypeStruct((e, 16), jnp.float32),
        ],
        mesh=mesh,
        scratch_types=[
            pltpu.VMEM((_EB,), jnp.int32),
            pltpu.VMEM((_EB,), jnp.int32),
            pltpu.VMEM((_EB, d), jnp.float32),
            pltpu.VMEM((_EB, d), jnp.float32),
            pltpu.VMEM((_EB, d), jnp.float32),
            pltpu.VMEM((_EB, d), jnp.float32),
            pltpu.VMEM((_EB, 16), jnp.float32),
            pltpu.SemaphoreType.DMA,
            pltpu.SemaphoreType.DMA,
            pltpu.SemaphoreType.DMA,
            pltpu.SemaphoreType.DMA,
        ],
    )
    def k(ha_h, hb_h, cw_h, row_h, col_h, g1_o, g2_o, cdr_o,
          idxr, idxc, g1, g2, crw, ccw, cdrb, s1, s2, s3, s4):
        wid = lax.axis_index("s") * _NC + lax.axis_index("c")

        def body(i, carry):
            base = wid * per_w + i * _EB
            pltpu.sync_copy(row_h.at[pl.ds(base, _EB)], idxr)
            pltpu.sync_copy(col_h.at[pl.ds(base, _EB)], idxc)
            a1 = pltpu.async_copy(ha_h.at[idxr], g1, s1)
            a2 = pltpu.async_copy(hb_h.at[idxc], g2, s2)
            a3 = pltpu.async_copy(cw_h.at[idxr], crw, s3)
            a4 = pltpu.async_copy(cw_h.at[idxc], ccw, s4)
            a3.wait()
            a4.wait()
            for ee in range(_EB):
                cdrb[ee, :] = crw[ee, pl.ds(0, 16)] - ccw[ee, pl.ds(0, 16)]
            a1.wait()
            a2.wait()
            pltpu.sync_copy(g1, g1_o.at[pl.ds(base, _EB)])
            pltpu.sync_copy(g2, g2_o.at[pl.ds(base, _EB)])
            pltpu.sync_copy(cdrb, cdr_o.at[pl.ds(base, _EB)])
            return carry

        lax.fori_loop(0, iters, body, 0)

    return k(ha, hb, cw, row, col)


def _sc_scatter(m, t16, row, n, zrow_d, zrow_16):
    e, d = m.shape
    per_w = e // _NW
    iters = per_w // _EB
    nz = 10
    per_z = n // nz
    mesh = plsc.VectorSubcoreMesh(core_axis_name="c", subcore_axis_name="s")

    @functools.partial(
        pl.kernel,
        out_type=[
            jax.ShapeDtypeStruct((_NC, n, d), jnp.float32),
            jax.ShapeDtypeStruct((_NC, n, 16), jnp.float32),
        ],
        mesh=mesh,
        scratch_types=[
            pltpu.VMEM_SHARED((n, d), jnp.float32),
            pltpu.VMEM_SHARED((n, 16), jnp.float32),
            pltpu.VMEM((_EB,), jnp.int32),
            pltpu.VMEM((_EB, d), jnp.float32),
            pltpu.VMEM((_EB, 16), jnp.float32),
        ],
    )
    def k(m_h, t_h, row_h, zd_h, z16_h, sm_o, st_o,
          acc_m, acc_t, idxr, mb, tb):
        cid = lax.axis_index("c")
        sid = lax.axis_index("s")
        wid = sid * _NC + cid

        @pl.when(sid < nz)
        def _zero():
            pltpu.sync_copy(zd_h, acc_m.at[pl.ds(sid * per_z, per_z)])
            pltpu.sync_copy(z16_h, acc_t.at[pl.ds(sid * per_z, per_z)])

        plsc.subcore_barrier()

        def body(i, carry):
            base = wid * per_w + i * _EB
            pltpu.sync_copy(row_h.at[pl.ds(base, _EB)], idxr)
            pltpu.sync_copy(m_h.at[pl.ds(base, _EB)], mb)
            pltpu.sync_copy(t_h.at[pl.ds(base, _EB)], tb)
            pltpu.sync_copy(mb, acc_m.at[idxr], add=True)
            pltpu.sync_copy(tb, acc_t.at[idxr], add=True)
            return carry

        lax.fori_loop(0, iters, body, 0)
        plsc.subcore_barrier()

        @pl.when(sid < nz)
        def _writeout():
            pltpu.sync_copy(acc_m.at[pl.ds(sid * per_z, per_z)],
                            sm_o.at[cid, pl.ds(sid * per_z, per_z)])
            pltpu.sync_copy(acc_t.at[pl.ds(sid * per_z, per_z)],
                            st_o.at[cid, pl.ds(sid * per_z, per_z)])

    return k(m, t16, row, zrow_d, zrow_16)


def _edge_body(g1_ref, g2_ref, cdr_ref, w1t_ref, cw0t_ref, cw1t_ref,
               consts_ref, m_out, t_out):
    cdr = cdr_ref[...]
    radial = jnp.sum(cdr * cdr, axis=1, keepdims=True)
    w0c = consts_ref[0:1, :]
    b1 = consts_ref[1:2, :]
    cb0 = consts_ref[2:3, :]
    cb1 = consts_ref[3:4, :]
    cw2 = consts_ref[4:5, :]
    pre = g1_ref[...] + g2_ref[...] + radial * w0c
    m0 = _silu(pre)
    m = _silu(jnp.dot(m0, w1t_ref[...], preferred_element_type=jnp.float32) + b1)
    t0 = _silu(jnp.dot(m, cw0t_ref[...], preferred_element_type=jnp.float32) + cb0)
    t1 = _silu(jnp.dot(t0, cw1t_ref[...], preferred_element_type=jnp.float32) + cb1)
    tt = jnp.sum(t1 * cw2, axis=1, keepdims=True)
    m_out[...] = m
    lane = jax.lax.broadcasted_iota(jnp.int32, cdr.shape, 1)
    t_out[...] = jnp.where(lane < 3, cdr * tt,
                           (lane == 3).astype(jnp.float32))


def _edge_mlp(g1, g2, cdr, w1t, cw0t, cw1t, consts, block=2000):
    e, d = g1.shape
    grid = e // block
    return pl.pallas_call(
        _edge_body,
        grid=(grid,),
        in_specs=[
            pl.BlockSpec((block, d), lambda i: (i, 0)),
            pl.BlockSpec((block, d), lambda i: (i, 0)),
            pl.BlockSpec((block, 16), lambda i: (i, 0)),
            pl.BlockSpec((d, d), lambda i: (0, 0)),
            pl.BlockSpec((d, d), lambda i: (0, 0)),
            pl.BlockSpec((d, d), lambda i: (0, 0)),
            pl.BlockSpec((8, d), lambda i: (0, 0)),
        ],
        out_specs=[
            pl.BlockSpec((block, d), lambda i: (i, 0)),
            pl.BlockSpec((block, 16), lambda i: (i, 0)),
        ],
        out_shape=[
            jax.ShapeDtypeStruct((e, d), jnp.float32),
            jax.ShapeDtypeStruct((e, 16), jnp.float32),
        ],
    )(g1, g2, cdr, w1t, cw0t, cw1t, consts)


def _node_body(h_ref, sm_ref, st_ref, c16_ref, w0at_ref, w0bt_ref, w1t_ref,
               w2t_ref, consts_ref, hout_ref, cout_ref):
    nb0 = consts_ref[0:1, :]
    nb1 = consts_ref[1:2, :]
    nb2 = consts_ref[2:3, :]
    st = st_ref[0] + st_ref[1]
    cnt = st[:, 3:4]
    inv = 1.0 / jnp.maximum(cnt, 1.0)
    agg = (sm_ref[0] + sm_ref[1]) * inv
    h = h_ref[...]
    a = _silu(jnp.dot(h, w0at_ref[...], preferred_element_type=jnp.float32)
              + jnp.dot(agg, w0bt_ref[...], preferred_element_type=jnp.float32)
              + nb0)
    a2 = _silu(jnp.dot(a, w1t_ref[...], preferred_element_type=jnp.float32) + nb1)
    outp = jnp.dot(a2, w2t_ref[...], preferred_element_type=jnp.float32) + nb2
    hout_ref[...] = h + outp
    cout_ref[...] = c16_ref[...] + st * inv


def _node_mlp(h, sm, st, c16, w0at, w0bt, w1t, w2t, consts, block=2000):
    n, d = h.shape
    grid = n // block
    return pl.pallas_call(
        _node_body,
        grid=(grid,),
        in_specs=[
            pl.BlockSpec((block, d), lambda i: (i, 0)),
            pl.BlockSpec((2, block, d), lambda i: (0, i, 0)),
            pl.BlockSpec((2, block, 16), lambda i: (0, i, 0)),
            pl.BlockSpec((block, 16), lambda i: (i, 0)),
            pl.BlockSpec((d, d), lambda i: (0, 0)),
            pl.BlockSpec((d, d), lambda i: (0, 0)),
            pl.BlockSpec((d, d), lambda i: (0, 0)),
            pl.BlockSpec((d, d), lambda i: (0, 0)),
            pl.BlockSpec((8, d), lambda i: (0, 0)),
        ],
        out_specs=[
            pl.BlockSpec((block, d), lambda i: (i, 0)),
            pl.BlockSpec((block, 16), lambda i: (i, 0)),
        ],
        out_shape=[
            jax.ShapeDtypeStruct((n, d), jnp.float32),
            jax.ShapeDtypeStruct((n, 16), jnp.float32),
        ],
    )(h, sm, st, c16, w0at, w0bt, w1t, w2t, consts)


def kernel(h, edge_index, coord, msg_W0, msg_b0, msg_W1, msg_b1,
           node_W0, node_b0, node_W1, node_b1, node_W2, node_b2,
           coord_W0, coord_b0, coord_W1, coord_b1, coord_W2):
    n, d = h.shape
    e = edge_index.shape[0]
    row = edge_index[:, 0]
    col = edge_index[:, 1]

    w0at = msg_W0[:, :d].T
    w0bt = msg_W0[:, d:2 * d].T
    w0c = msg_W0[:, 2 * d]
    zeros_row = jnp.zeros((1, d), jnp.float32)
    edge_consts = jnp.concatenate([
        w0c[None, :], msg_b1[None, :], coord_b0[None, :], coord_b1[None, :],
        coord_W2, zeros_row, zeros_row, zeros_row], axis=0)
    node_consts = jnp.concatenate([
        node_b0[None, :], node_b1[None, :], node_b2[None, :],
        zeros_row, zeros_row, zeros_row, zeros_row, zeros_row], axis=0)

    coord16 = jnp.pad(coord, ((0, 0), (0, 16 - coord.shape[1])))
    coordwide = jnp.pad(coord, ((0, 0), (0, d - coord.shape[1])))

    ha, hb = _node_proj(h, w0at, w0bt, msg_b0[None, :])

    g1, g2, cdr = _sc_gather(ha, hb, coordwide, row, col)

    m, t16 = _edge_mlp(g1, g2, cdr, msg_W1.T, coord_W0.T, coord_W1.T,
                       edge_consts)

    if True:
        sm = jax.ops.segment_sum(m, row, num_segments=n)
        st = jax.ops.segment_sum(t16, row, num_segments=n)
        sm2 = jnp.stack([sm, jnp.zeros_like(sm)])
        st2 = jnp.stack([st, jnp.zeros_like(st)])
    else:
        zrow_d = jnp.zeros((n // 10, d), jnp.float32)
        zrow_16 = jnp.zeros((n // 10, 16), jnp.float32)
        sm2, st2 = _sc_scatter(m, t16, row, n, zrow_d, zrow_16)

    h_out, c16_out = _node_mlp(h, sm2, st2, coord16, node_W0[:, :d].T,
                               node_W0[:, d:].T, node_W1.T, node_W2.T,
                               node_consts)
    return (h_out, c16_out[:, :3])


# speedup vs baseline: 3.6925x; 1.3418x over previous
import functools

import jax
import jax.numpy as jnp
from jax import lax
from jax.experimental import pallas as pl
from jax.experimental.pallas import tpu as pltpu
from jax.experimental.pallas import tpu_sc as plsc

_NC = 2
_NS = 16
_NW = _NC * _NS
_EB = 80


def _silu(x):
    return x * jax.nn.sigmoid(x)


def _proj_body(h_ref, w0at_ref, w0bt_ref, b0_ref, ha_ref, hb_ref):
    h = h_ref[...]
    ha_ref[...] = jnp.dot(h, w0at_ref[...], preferred_element_type=jnp.float32) + b0_ref[...]
    hb_ref[...] = jnp.dot(h, w0bt_ref[...], preferred_element_type=jnp.float32)


def _node_proj(h, w0at, w0bt, b0row, block=2000):
    n, d = h.shape
    grid = n // block
    return pl.pallas_call(
        _proj_body,
        grid=(grid,),
        in_specs=[
            pl.BlockSpec((block, d), lambda i: (i, 0)),
            pl.BlockSpec((d, d), lambda i: (0, 0)),
            pl.BlockSpec((d, d), lambda i: (0, 0)),
            pl.BlockSpec((1, d), lambda i: (0, 0)),
        ],
        out_specs=[
            pl.BlockSpec((block, d), lambda i: (i, 0)),
            pl.BlockSpec((block, d), lambda i: (i, 0)),
        ],
        out_shape=[
            jax.ShapeDtypeStruct((n, d), jnp.float32),
            jax.ShapeDtypeStruct((n, d), jnp.float32),
        ],
    )(h, w0at, w0bt, b0row)


def _sc_gather(ha, hb, cw, row, col):
    e = row.shape[0]
    d = ha.shape[1]
    per_w = e // _NW
    iters = per_w // _EB
    mesh = plsc.VectorSubcoreMesh(core_axis_name="c", subcore_axis_name="s")

    @functools.partial(
        pl.kernel,
        out_type=[
            jax.ShapeDtypeStruct((e, d), jnp.float32),
            jax.ShapeDtypeStruct((e, d), jnp.float32),
            jax.ShapeDtypeStruct((e, 16), jnp.float32),
        ],
        mesh=mesh,
        scratch_types=[
            pltpu.VMEM((_EB,), jnp.int32),
            pltpu.VMEM((_EB,), jnp.int32),
            pltpu.VMEM((_EB, d), jnp.float32),
            pltpu.VMEM((_EB, d), jnp.float32),
            pltpu.VMEM((_EB, d), jnp.float32),
            pltpu.VMEM((_EB, d), jnp.float32),
            pltpu.VMEM((_EB, 16), jnp.float32),
            pltpu.SemaphoreType.DMA,
            pltpu.SemaphoreType.DMA,
            pltpu.SemaphoreType.DMA,
            pltpu.SemaphoreType.DMA,
        ],
    )
    def k(ha_h, hb_h, cw_h, row_h, col_h, g1_o, g2_o, cdr_o,
          idxr, idxc, g1, g2, crw, ccw, cdrb, s1, s2, s3, s4):
        wid = lax.axis_index("s") * _NC + lax.axis_index("c")

        def body(i, carry):
            base = wid * per_w + i * _EB
            pltpu.sync_copy(row_h.at[pl.ds(base, _EB)], idxr)
            pltpu.sync_copy(col_h.at[pl.ds(base, _EB)], idxc)
            a1 = pltpu.async_copy(ha_h.at[idxr], g1, s1)
            a2 = pltpu.async_copy(hb_h.at[idxc], g2, s2)
            a3 = pltpu.async_copy(cw_h.at[idxr], crw, s3)
            a4 = pltpu.async_copy(cw_h.at[idxc], ccw, s4)
            a3.wait()
            a4.wait()
            for ee in range(_EB):
                cdrb[ee, :] = crw[ee, pl.ds(0, 16)] - ccw[ee, pl.ds(0, 16)]
            a1.wait()
            a2.wait()
            pltpu.sync_copy(g1, g1_o.at[pl.ds(base, _EB)])
            pltpu.sync_copy(g2, g2_o.at[pl.ds(base, _EB)])
            pltpu.sync_copy(cdrb, cdr_o.at[pl.ds(base, _EB)])
            return carry

        lax.fori_loop(0, iters, body, 0)

    return k(ha, hb, cw, row, col)


def _sc_scatter_m(dat, row2, n, zrow):
    e = dat.shape[0]
    d = dat.shape[1]
    half = n // _NC
    per_s = e // _NS
    iters = per_s // _EB
    nz = 5
    per_z = half // nz
    nq = 5
    mesh = plsc.VectorSubcoreMesh(core_axis_name="c", subcore_axis_name="s")

    @functools.partial(
        pl.kernel,
        out_type=jax.ShapeDtypeStruct((n, d), jnp.float32),
        mesh=mesh,
        scratch_types=[
            pltpu.VMEM_SHARED((half + 8, d), jnp.float32),
            pltpu.VMEM((_EB,), jnp.int32),
            pltpu.VMEM((_EB, d), jnp.float32),
            pltpu.VMEM((per_z // nq, d), jnp.float32),
        ],
    )
    def k(d_h, row_h, z_h, s_o, acc, idxr, db, bb):
        cid = lax.axis_index("c")
        sid = lax.axis_index("s")
        lo = cid * half

        @pl.when(sid < nz)
        def _zero():
            pltpu.sync_copy(z_h, bb)
            for q in range(nq):
                pltpu.sync_copy(
                    bb, acc.at[pl.ds(sid * per_z + q * (per_z // nq),
                                     per_z // nq)])

        plsc.subcore_barrier()

        def body(i, carry):
            base = sid * per_s + i * _EB
            pltpu.sync_copy(row_h.at[pl.ds(cid * e + base, _EB)], idxr)
            pltpu.sync_copy(d_h.at[pl.ds(base, _EB)], db)
            pltpu.sync_copy(db, acc.at[idxr], add=True)
            return carry

        lax.fori_loop(0, iters, body, 0)
        plsc.subcore_barrier()

        @pl.when(sid < nz)
        def _writeout():
            for q in range(nq):
                off = sid * per_z + q * (per_z // nq)
                pltpu.sync_copy(acc.at[pl.ds(off, per_z // nq)], bb)
                pltpu.sync_copy(bb, s_o.at[pl.ds(lo + off, per_z // nq)])

    return k(dat, row2, zrow)


def _sc_scatter_t(dat, row, n, zrow, d):
    e, w = dat.shape
    per_w = e // _NW
    iters = per_w // _EB
    nz = 10
    per_z = n // nz
    nq = 5
    mesh = plsc.VectorSubcoreMesh(core_axis_name="c", subcore_axis_name="s")

    @functools.partial(
        pl.kernel,
        out_type=jax.ShapeDtypeStruct((_NC, n, d), jnp.float32),
        mesh=mesh,
        scratch_types=[
            pltpu.VMEM_SHARED((n, d), jnp.float32),
            pltpu.VMEM((_EB,), jnp.int32),
            pltpu.VMEM((_EB, w), jnp.float32),
            pltpu.VMEM((_EB, d), jnp.float32),
            pltpu.VMEM((per_z // nq, d), jnp.float32),
        ],
    )
    def k(d_h, row_h, z_h, s_o, acc, idxr, tb, db, bb):
        cid = lax.axis_index("c")
        sid = lax.axis_index("s")
        wid = sid * _NC + cid

        zv = jnp.zeros((16,), jnp.float32)
        for r in range(_EB):
            for q in range(d // 16):
                db[r, pl.ds(q * 16, 16)] = zv

        @pl.when(sid < nz)
        def _zero():
            pltpu.sync_copy(z_h, bb)
            for q in range(nq):
                pltpu.sync_copy(
                    bb, acc.at[pl.ds(sid * per_z + q * (per_z // nq),
                                     per_z // nq)])

        plsc.subcore_barrier()

        def body(i, carry):
            base = wid * per_w + i * _EB
            pltpu.sync_copy(row_h.at[pl.ds(base, _EB)], idxr)
            pltpu.sync_copy(d_h.at[pl.ds(base, _EB)], tb)
            for r in range(_EB):
                db[r, pl.ds(0, 16)] = tb[r, :]
            pltpu.sync_copy(db, acc.at[idxr], add=True)
            return carry

        lax.fori_loop(0, iters, body, 0)
        plsc.subcore_barrier()

        @pl.when(sid < nz)
        def _writeout():
            for q in range(nq):
                sl = pl.ds(sid * per_z + q * (per_z // nq), per_z // nq)
                pltpu.sync_copy(acc.at[sl], bb)
                pltpu.sync_copy(bb, s_o.at[cid, sl])

    return k(dat, row, zrow)


def _edge_body(g1_ref, g2_ref, cdr_ref, w1t_ref, cw0t_ref, cw1t_ref,
               consts_ref, m_out, t_out):
    cdr = cdr_ref[...]
    radial = jnp.sum(cdr * cdr, axis=1, keepdims=True)
    w0c = consts_ref[0:1, :]
    b1 = consts_ref[1:2, :]
    cb0 = consts_ref[2:3, :]
    cb1 = consts_ref[3:4, :]
    cw2 = consts_ref[4:5, :]
    pre = g1_ref[...] + g2_ref[...] + radial * w0c
    m0 = _silu(pre)
    m = _silu(jnp.dot(m0, w1t_ref[...], preferred_element_type=jnp.float32) + b1)
    t0 = _silu(jnp.dot(m, cw0t_ref[...], preferred_element_type=jnp.float32) + cb0)
    t1 = _silu(jnp.dot(t0, cw1t_ref[...], preferred_element_type=jnp.float32) + cb1)
    tt = jnp.sum(t1 * cw2, axis=1, keepdims=True)
    m_out[...] = m
    lane = jax.lax.broadcasted_iota(jnp.int32, cdr.shape, 1)
    t_out[...] = jnp.where(lane < 3, cdr * tt,
                           (lane == 3).astype(jnp.float32))


def _edge_mlp(g1, g2, cdr, w1t, cw0t, cw1t, consts, block=2000):
    e, d = g1.shape
    grid = e // block
    return pl.pallas_call(
        _edge_body,
        grid=(grid,),
        in_specs=[
            pl.BlockSpec((block, d), lambda i: (i, 0)),
            pl.BlockSpec((block, d), lambda i: (i, 0)),
            pl.BlockSpec((block, 16), lambda i: (i, 0)),
            pl.BlockSpec((d, d), lambda i: (0, 0)),
            pl.BlockSpec((d, d), lambda i: (0, 0)),
            pl.BlockSpec((d, d), lambda i: (0, 0)),
            pl.BlockSpec((8, d), lambda i: (0, 0)),
        ],
        out_specs=[
            pl.BlockSpec((block, d), lambda i: (i, 0)),
            pl.BlockSpec((block, 16), lambda i: (i, 0)),
        ],
        out_shape=[
            jax.ShapeDtypeStruct((e, d), jnp.float32),
            jax.ShapeDtypeStruct((e, 16), jnp.float32),
        ],
    )(g1, g2, cdr, w1t, cw0t, cw1t, consts)


def _node_body(h_ref, sm_ref, st_ref, c16_ref, w0at_ref, w0bt_ref, w1t_ref,
               w2t_ref, consts_ref, hout_ref, cout_ref):
    nb0 = consts_ref[0:1, :]
    nb1 = consts_ref[1:2, :]
    nb2 = consts_ref[2:3, :]
    st = st_ref[0] + st_ref[1]
    cnt = st[:, 3:4]
    inv = 1.0 / jnp.maximum(cnt, 1.0)
    agg = sm_ref[...] * inv
    h = h_ref[...]
    a = _silu(jnp.dot(h, w0at_ref[...], preferred_element_type=jnp.float32)
              + jnp.dot(agg, w0bt_ref[...], preferred_element_type=jnp.float32)
              + nb0)
    a2 = _silu(jnp.dot(a, w1t_ref[...], preferred_element_type=jnp.float32) + nb1)
    outp = jnp.dot(a2, w2t_ref[...], preferred_element_type=jnp.float32) + nb2
    hout_ref[...] = h + outp
    cout_ref[...] = c16_ref[...] + st[:, :16] * inv


def _node_mlp(h, sm, st, c16, w0at, w0bt, w1t, w2t, consts, block=2000):
    n, d = h.shape
    grid = n // block
    return pl.pallas_call(
        _node_body,
        grid=(grid,),
        in_specs=[
            pl.BlockSpec((block, d), lambda i: (i, 0)),
            pl.BlockSpec((block, d), lambda i: (i, 0)),
            pl.BlockSpec((2, block, d), lambda i: (0, i, 0)),
            pl.BlockSpec((block, 16), lambda i: (i, 0)),
            pl.BlockSpec((d, d), lambda i: (0, 0)),
            pl.BlockSpec((d, d), lambda i: (0, 0)),
            pl.BlockSpec((d, d), lambda i: (0, 0)),
            pl.BlockSpec((d, d), lambda i: (0, 0)),
            pl.BlockSpec((8, d), lambda i: (0, 0)),
        ],
        out_specs=[
            pl.BlockSpec((block, d), lambda i: (i, 0)),
            pl.BlockSpec((block, 16), lambda i: (i, 0)),
        ],
        out_shape=[
            jax.ShapeDtypeStruct((n, d), jnp.float32),
            jax.ShapeDtypeStruct((n, 16), jnp.float32),
        ],
    )(h, sm, st, c16, w0at, w0bt, w1t, w2t, consts)


def kernel(h, edge_index, coord, msg_W0, msg_b0, msg_W1, msg_b1,
           node_W0, node_b0, node_W1, node_b1, node_W2, node_b2,
           coord_W0, coord_b0, coord_W1, coord_b1, coord_W2):
    n, d = h.shape
    row = edge_index[:, 0]
    col = edge_index[:, 1]

    w0at = msg_W0[:, :d].T
    w0bt = msg_W0[:, d:2 * d].T
    w0c = msg_W0[:, 2 * d]
    zeros_row = jnp.zeros((1, d), jnp.float32)
    edge_consts = jnp.concatenate([
        w0c[None, :], msg_b1[None, :], coord_b0[None, :], coord_b1[None, :],
        coord_W2, zeros_row, zeros_row, zeros_row], axis=0)
    node_consts = jnp.concatenate([
        node_b0[None, :], node_b1[None, :], node_b2[None, :],
        zeros_row, zeros_row, zeros_row, zeros_row, zeros_row], axis=0)

    coord16 = jnp.pad(coord, ((0, 0), (0, 16 - coord.shape[1])))
    coordwide = jnp.pad(coord, ((0, 0), (0, d - coord.shape[1])))

    ha, hb = _node_proj(h, w0at, w0bt, msg_b0[None, :])

    g1, g2, cdr = _sc_gather(ha, hb, coordwide, row, col)

    m, t16 = _edge_mlp(g1, g2, cdr, msg_W1.T, coord_W0.T, coord_W1.T,
                       edge_consts)

    zrow_d = jnp.zeros((200, d), jnp.float32)
    half = n // _NC
    row_a = jnp.where(row < half, row, half)
    row_b = jnp.where(row >= half, row - half, half)
    row2 = jnp.concatenate([row_a, row_b])
    sm = _sc_scatter_m(m, row2, n, zrow_d)
    st2 = _sc_scatter_t(t16, row, n, zrow_d, d)

    h_out, c16_out = _node_mlp(h, sm, st2, coord16, node_W0[:, :d].T,
                               node_W0[:, d:].T, node_W1.T, node_W2.T,
                               node_consts)
    return (h_out, c16_out[:, :3])
